# baseline (speedup 1.0000x reference)
import numpy as np
import jax
import jax.numpy as jnp
from jax.experimental import pallas as pl
from jax.experimental.pallas import tpu as pltpu

C1 = 32
C2 = 16


def _quad_input(x):
    B = x.shape[0]
    xp = jnp.pad(x[:, 0], ((0, 0), (1, 3), (1, 3)))
    xq = xp.reshape(B, 8, 4, 8, 4).transpose(0, 1, 3, 2, 4)
    return xq.reshape(B, 8, 8, 16)


def _conv1_effective_weight(conv1_w):
    sel = np.zeros((36, 16, 9), dtype=np.float32)
    for rp in (0, 1):
        for cp in (0, 1):
            for dy in (0, 1):
                for dx in (0, 1):
                    g = ((rp * 2 + cp) * 2 + dy) * 2 + dx
                    for ky in range(3):
                        for kx in range(3):
                            A = 2 * rp + dy + ky
                            Bc = 2 * cp + dx + kx
                            sel[A * 6 + Bc, g, ky * 3 + kx] = 1.0
    w1k = conv1_w.reshape(C1, 9).T
    w = jnp.einsum("tgk,kc->tgc", jnp.asarray(sel), w1k,
                   precision=jax.lax.Precision.HIGHEST)
    w = w.reshape(36, 16 * C1)
    qsel = np.zeros((4, 16, 36), dtype=np.float32)
    for dA in (0, 1):
        for dE in (0, 1):
            for a in range(4):
                for em in range(4):
                    A, e = 4 * dA + a, 4 * dE + em
                    if A < 6 and e < 6:
                        qsel[dA * 2 + dE, a * 4 + em, A * 6 + e] = 1.0
    wq = jnp.einsum("qst,tm->qsm", jnp.asarray(qsel), w,
                    precision=jax.lax.Precision.HIGHEST)
    return wq.reshape(64, 16 * C1)


def _conv2_grouped_weight(conv2_w):
    sel = np.zeros((16, 4, 9), dtype=np.float32)
    for u in range(4):
        for v in range(4):
            for dy in (0, 1):
                for dx in (0, 1):
                    ky, kx = u - dy, v - dx
                    if 0 <= ky < 3 and 0 <= kx < 3:
                        sel[u * 4 + v, dy * 2 + dx, ky * 3 + kx] = 1.0
    w2k = jnp.transpose(conv2_w, (2, 3, 1, 0)).reshape(9, C1, C2)
    w = jnp.einsum("sgk,kcd->scgd", jnp.asarray(sel), w2k,
                   precision=jax.lax.Precision.HIGHEST)
    return w.reshape(16 * C1, 4 * C2)


def _fc1_weight_nhwc(fc1_w):
    ii, jj, cc = np.meshgrid(np.arange(7), np.arange(7), np.arange(C2),
                             indexing="ij")
    perm = (cc * 49 + ii * 7 + jj).reshape(-1)
    return fc1_w[jnp.asarray(perm), :]


def _fused_body(xq_ref, w1_ref, b1_ref, w2_ref, b2_ref,
                f1w_ref, f1b_ref, f2w_ref, f2b_ref, out_ref, p1_scr,
                flat_scr):
    TB = xq_ref.shape[0]

    xq = xq_ref[...]
    z1 = None
    for dA in (0, 1):
        for dE in (0, 1):
            q = dA * 2 + dE
            t = xq[:, dA:dA + 7, dE:dE + 7, :].reshape(TB * 49, 16)
            part = jnp.dot(t, w1_ref[q * 16:(q + 1) * 16, :],
                           preferred_element_type=jnp.float32)
            z1 = part if z1 is None else z1 + part
    z1 = jnp.maximum(z1 + b1_ref[...], 0.0)
    p1_scr[...] = jnp.zeros_like(p1_scr)
    for rp in (0, 1):
        for cp in (0, 1):
            best = None
            for dy in (0, 1):
                for dx in (0, 1):
                    g = ((rp * 2 + cp) * 2 + dy) * 2 + dx
                    acc = z1[:, g * C1:(g + 1) * C1]
                    best = acc if best is None else jnp.maximum(best, acc)
            p1_scr[(1 - rp) * 2 + (1 - cp), :, rp:rp + 7, cp:cp + 7, :] = (
                best.reshape(TB, 7, 7, C1))

    q2 = jnp.concatenate(
        [p1_scr[(u % 2) * 2 + (v % 2), :,
                u // 2:u // 2 + 7, v // 2:v // 2 + 7, :]
         for u in range(4) for v in range(4)], axis=-1)
    z2 = jnp.dot(q2.reshape(TB * 49, 16 * C1), w2_ref[...],
                 preferred_element_type=jnp.float32)
    z2 = jnp.maximum(z2 + b2_ref[...], 0.0)
    p2 = jnp.maximum(jnp.maximum(z2[:, 0 * C2:1 * C2], z2[:, 1 * C2:2 * C2]),
                     jnp.maximum(z2[:, 2 * C2:3 * C2], z2[:, 3 * C2:4 * C2]))

    p2r = p2.reshape(TB, 49, C2)
    for s in range(49):
        flat_scr[:, s * C2:(s + 1) * C2] = p2r[:, s, :]
    h = jnp.dot(flat_scr[...], f1w_ref[...],
                preferred_element_type=jnp.float32)
    h = jnp.maximum(h + f1b_ref[...], 0.0)
    out_ref[...] = (jnp.dot(h, f2w_ref[...],
                            preferred_element_type=jnp.float32) + f2b_ref[...])


def kernel(x, conv1_w, conv1_b, conv2_w, conv2_b, fc1_w, fc1_b, fc2_w, fc2_b):
    B = x.shape[0]
    TB = 64
    while B % TB:
        TB //= 2

    xq = _quad_input(x)
    w1e = _conv1_effective_weight(conv1_w)
    b1c = jnp.tile(conv1_b.reshape(1, C1), (1, 16))
    w2g = _conv2_grouped_weight(conv2_w)
    b2c = jnp.tile(conv2_b.reshape(1, C2), (1, 4))
    f1w = _fc1_weight_nhwc(fc1_w)

    return pl.pallas_call(
        _fused_body,
        out_shape=jax.ShapeDtypeStruct((B, 10), jnp.float32),
        grid=(B // TB,),
        in_specs=[
            pl.BlockSpec((TB, 8, 8, 16), lambda b: (b, 0, 0, 0)),
            pl.BlockSpec((64, 16 * C1), lambda b: (0, 0)),
            pl.BlockSpec((1, 16 * C1), lambda b: (0, 0)),
            pl.BlockSpec((16 * C1, 4 * C2), lambda b: (0, 0)),
            pl.BlockSpec((1, 4 * C2), lambda b: (0, 0)),
            pl.BlockSpec((49 * C2, 32), lambda b: (0, 0)),
            pl.BlockSpec((1, 32), lambda b: (0, 0)),
            pl.BlockSpec((32, 10), lambda b: (0, 0)),
            pl.BlockSpec((1, 10), lambda b: (0, 0)),
        ],
        out_specs=pl.BlockSpec((TB, 10), lambda b: (b, 0)),
        scratch_shapes=[pltpu.VMEM((4, TB, 8, 8, C1), jnp.float32),
                        pltpu.VMEM((TB, 49 * C2), jnp.float32)],
        compiler_params=pltpu.CompilerParams(
            dimension_semantics=("parallel",),
            vmem_limit_bytes=64 * 1024 * 1024,
        ),
    )(xq, w1e, b1c, w2g, b2c,
      f1w, fc1_b.reshape(1, 32), fc2_w, fc2_b.reshape(1, 10))


# speedup vs baseline: 1.0218x; 1.0218x over previous
import numpy as np
import jax
import jax.numpy as jnp
from jax.experimental import pallas as pl
from jax.experimental.pallas import tpu as pltpu

C1 = 32
C2 = 16


def _quad_input(x):
    B = x.shape[0]
    xp = jnp.pad(x[:, 0], ((0, 0), (1, 3), (1, 3)))
    return xp.reshape(B, 8, 8, 16)


def _conv1_effective_weight(conv1_w):
    sel = np.zeros((36, 16, 9), dtype=np.float32)
    for rp in (0, 1):
        for cp in (0, 1):
            for dy in (0, 1):
                for dx in (0, 1):
                    g = ((rp * 2 + cp) * 2 + dy) * 2 + dx
                    for ky in range(3):
                        for kx in range(3):
                            A = 2 * rp + dy + ky
                            Bc = 2 * cp + dx + kx
                            sel[A * 6 + Bc, g, ky * 3 + kx] = 1.0
    w1k = conv1_w.reshape(C1, 9).T
    w = jnp.einsum("tgk,kc->tgc", jnp.asarray(sel), w1k,
                   precision=jax.lax.Precision.HIGHEST)
    w = w.reshape(36, 16 * C1)
    qsel = np.zeros((4, 16, 36), dtype=np.float32)
    for dA in (0, 1):
        for dE in (0, 1):
            for a in range(4):
                for em in range(4):
                    A, e = 4 * dA + a, 4 * dE + em
                    if A < 6 and e < 6:
                        qsel[dA * 2 + dE, a * 4 + em, A * 6 + e] = 1.0
    wq = jnp.einsum("qst,tm->qsm", jnp.asarray(qsel), w,
                    precision=jax.lax.Precision.HIGHEST)
    return wq.reshape(64, 16 * C1)


def _conv2_grouped_weight(conv2_w):
    sel = np.zeros((16, 4, 9), dtype=np.float32)
    for u in range(4):
        for v in range(4):
            for dy in (0, 1):
                for dx in (0, 1):
                    ky, kx = u - dy, v - dx
                    if 0 <= ky < 3 and 0 <= kx < 3:
                        sel[u * 4 + v, dy * 2 + dx, ky * 3 + kx] = 1.0
    w2k = jnp.transpose(conv2_w, (2, 3, 1, 0)).reshape(9, C1, C2)
    w = jnp.einsum("sgk,kcd->scgd", jnp.asarray(sel), w2k,
                   precision=jax.lax.Precision.HIGHEST)
    return w.reshape(16 * C1, 4 * C2)


def _fc1_weight_nhwc(fc1_w):
    ii, jj, cc = np.meshgrid(np.arange(7), np.arange(7), np.arange(C2),
                             indexing="ij")
    perm = (cc * 49 + ii * 7 + jj).reshape(-1)
    return fc1_w[jnp.asarray(perm), :]


def _fused_body(xq_ref, w1_ref, b1_ref, w2_ref, b2_ref,
                f1w_ref, f1b_ref, f2w_ref, f2b_ref, out_ref, p1_scr,
                flat_scr):
    TB = xq_ref.shape[0]

    xq = xq_ref[...]
    z1 = None
    for dA in (0, 1):
        for dE in (0, 1):
            q = dA * 2 + dE
            t = xq[:, dA:dA + 7, dE:dE + 7, :].reshape(TB * 49, 16)
            part = jnp.dot(t, w1_ref[q * 16:(q + 1) * 16, :],
                           preferred_element_type=jnp.float32)
            z1 = part if z1 is None else z1 + part
    z1 = jnp.maximum(z1 + b1_ref[...], 0.0)
    p1_scr[...] = jnp.zeros_like(p1_scr)
    for rp in (0, 1):
        for cp in (0, 1):
            best = None
            for dy in (0, 1):
                for dx in (0, 1):
                    g = ((rp * 2 + cp) * 2 + dy) * 2 + dx
                    acc = z1[:, g * C1:(g + 1) * C1]
                    best = acc if best is None else jnp.maximum(best, acc)
            p1_scr[(1 - rp) * 2 + (1 - cp), :, rp:rp + 7, cp:cp + 7, :] = (
                best.reshape(TB, 7, 7, C1))

    q2 = jnp.concatenate(
        [p1_scr[(u % 2) * 2 + (v % 2), :,
                u // 2:u // 2 + 7, v // 2:v // 2 + 7, :]
         for u in range(4) for v in range(4)], axis=-1)
    z2 = jnp.dot(q2.reshape(TB * 49, 16 * C1), w2_ref[...],
                 preferred_element_type=jnp.float32)
    z2 = jnp.maximum(z2 + b2_ref[...], 0.0)
    p2 = jnp.maximum(jnp.maximum(z2[:, 0 * C2:1 * C2], z2[:, 1 * C2:2 * C2]),
                     jnp.maximum(z2[:, 2 * C2:3 * C2], z2[:, 3 * C2:4 * C2]))

    p2r = p2.reshape(TB, 49, C2)
    for s in range(49):
        flat_scr[:, s * C2:(s + 1) * C2] = p2r[:, s, :]
    h = jnp.dot(flat_scr[...], f1w_ref[...],
                preferred_element_type=jnp.float32)
    h = jnp.maximum(h + f1b_ref[...], 0.0)
    out_ref[...] = (jnp.dot(h, f2w_ref[...],
                            preferred_element_type=jnp.float32) + f2b_ref[...])


def kernel(x, conv1_w, conv1_b, conv2_w, conv2_b, fc1_w, fc1_b, fc2_w, fc2_b):
    B = x.shape[0]
    TB = 64
    while B % TB:
        TB //= 2

    xq = _quad_input(x)
    w1e = _conv1_effective_weight(conv1_w)
    b1c = jnp.tile(conv1_b.reshape(1, C1), (1, 16))
    w2g = _conv2_grouped_weight(conv2_w)
    b2c = jnp.tile(conv2_b.reshape(1, C2), (1, 4))
    f1w = _fc1_weight_nhwc(fc1_w)

    return pl.pallas_call(
        _fused_body,
        out_shape=jax.ShapeDtypeStruct((B, 10), jnp.float32),
        grid=(B // TB,),
        in_specs=[
            pl.BlockSpec((TB, 8, 8, 16), lambda b: (b, 0, 0, 0)),
            pl.BlockSpec((64, 16 * C1), lambda b: (0, 0)),
            pl.BlockSpec((1, 16 * C1), lambda b: (0, 0)),
            pl.BlockSpec((16 * C1, 4 * C2), lambda b: (0, 0)),
            pl.BlockSpec((1, 4 * C2), lambda b: (0, 0)),
            pl.BlockSpec((49 * C2, 32), lambda b: (0, 0)),
            pl.BlockSpec((1, 32), lambda b: (0, 0)),
            pl.BlockSpec((32, 10), lambda b: (0, 0)),
            pl.BlockSpec((1, 10), lambda b: (0, 0)),
        ],
        out_specs=pl.BlockSpec((TB, 10), lambda b: (b, 0)),
        scratch_shapes=[pltpu.VMEM((4, TB, 8, 8, C1), jnp.float32),
                        pltpu.VMEM((TB, 49 * C2), jnp.float32)],
        compiler_params=pltpu.CompilerParams(
            dimension_semantics=("parallel",),
            vmem_limit_bytes=64 * 1024 * 1024,
        ),
    )(xq, w1e, b1c, w2g, b2c,
      f1w, fc1_b.reshape(1, 32), fc2_w, fc2_b.reshape(1, 10))


# speedup vs baseline: 2.1729x; 2.1267x over previous
import numpy as np
import jax
import jax.numpy as jnp
from jax.experimental import pallas as pl
from jax.experimental.pallas import tpu as pltpu

C1 = 32
C2 = 16


def _quad_input(x):
    B = x.shape[0]
    xp = jnp.pad(x[:, 0], ((0, 0), (1, 3), (1, 3)))
    xq = xp.reshape(B, 8, 4, 8, 4).transpose(1, 3, 0, 2, 4)
    return xq.reshape(8, 8, B, 16)


def _conv1_effective_weight(conv1_w):
    sel = np.zeros((36, 16, 9), dtype=np.float32)
    for rp in (0, 1):
        for cp in (0, 1):
            for dy in (0, 1):
                for dx in (0, 1):
                    g = (dy * 2 + dx) * 4 + (rp * 2 + cp)
                    for ky in range(3):
                        for kx in range(3):
                            A = 2 * rp + dy + ky
                            e = 2 * cp + dx + kx
                            sel[A * 6 + e, g, ky * 3 + kx] = 1.0
    w1k = conv1_w.reshape(C1, 9).T
    w = jnp.einsum("tgk,kc->tgc", jnp.asarray(sel), w1k,
                   precision=jax.lax.Precision.HIGHEST)
    w = w.reshape(36, 16 * C1)
    qsel = np.zeros((4, 16, 36), dtype=np.float32)
    for dA in (0, 1):
        for dE in (0, 1):
            for a in range(4):
                for em in range(4):
                    A, e = 4 * dA + a, 4 * dE + em
                    if A < 6 and e < 6:
                        qsel[dA * 2 + dE, a * 4 + em, A * 6 + e] = 1.0
    wq = jnp.einsum("qst,tm->qsm", jnp.asarray(qsel), w,
                    precision=jax.lax.Precision.HIGHEST)
    return wq.reshape(64, 16 * C1)


def _conv2_grouped_weight(conv2_w):
    sel = np.zeros((16, 4, 9), dtype=np.float32)
    for u in range(4):
        for v in range(4):
            for dy in (0, 1):
                for dx in (0, 1):
                    ky, kx = u - dy, v - dx
                    if 0 <= ky < 3 and 0 <= kx < 3:
                        sel[u * 4 + v, dy * 2 + dx, ky * 3 + kx] = 1.0
    w2k = jnp.transpose(conv2_w, (2, 3, 1, 0)).reshape(9, C1, C2)
    w = jnp.einsum("sgk,kcd->scgd", jnp.asarray(sel), w2k,
                   precision=jax.lax.Precision.HIGHEST)
    return w.reshape(16 * C1, 4 * C2)


def _fc1_weight_nhwc(fc1_w):
    ii, jj, cc = np.meshgrid(np.arange(7), np.arange(7), np.arange(C2),
                             indexing="ij")
    perm = (cc * 49 + ii * 7 + jj).reshape(-1)
    return fc1_w[jnp.asarray(perm), :]


def _fused_body(xq_ref, w1_ref, b1_ref, w2_ref, b2_ref,
                f1w_ref, f1b_ref, f2w_ref, f2b_ref, out_ref, p1_scr,
                flat_scr):
    TB = xq_ref.shape[2]
    M = 49 * TB

    @pl.when(pl.program_id(0) == 0)
    def _init():
        p1_scr[...] = jnp.zeros_like(p1_scr)

    xq = xq_ref[...]
    z1 = None
    for dA in (0, 1):
        for dE in (0, 1):
            q = dA * 2 + dE
            t = xq[dA:dA + 7, dE:dE + 7, :, :].reshape(M, 16)
            part = jnp.dot(t, w1_ref[q * 16:(q + 1) * 16, :],
                           preferred_element_type=jnp.float32)
            z1 = part if z1 is None else z1 + part
    z1 = jnp.maximum(z1 + b1_ref[...], 0.0)
    best4 = jnp.maximum(jnp.maximum(z1[:, 0:128], z1[:, 128:256]),
                        jnp.maximum(z1[:, 256:384], z1[:, 384:512]))
    for rp in (0, 1):
        for cp in (0, 1):
            best = best4[:, (rp * 2 + cp) * 32:(rp * 2 + cp + 1) * 32]
            p1_scr[(1 - rp) * 2 + (1 - cp), rp:rp + 7, cp:cp + 7, :, :] = (
                best.reshape(7, 7, TB, C1))

    q2 = jnp.concatenate(
        [p1_scr[(u % 2) * 2 + (v % 2),
                u // 2:u // 2 + 7, v // 2:v // 2 + 7, :, :]
         for u in range(4) for v in range(4)], axis=-1)
    z2 = jnp.dot(q2.reshape(M, 16 * C1), w2_ref[...],
                 preferred_element_type=jnp.float32)
    z2 = jnp.maximum(z2 + b2_ref[...], 0.0)
    p2 = jnp.maximum(jnp.maximum(z2[:, 0 * C2:1 * C2], z2[:, 1 * C2:2 * C2]),
                     jnp.maximum(z2[:, 2 * C2:3 * C2], z2[:, 3 * C2:4 * C2]))

    p2r = p2.reshape(7, 7, TB, C2)
    for i in range(7):
        for j in range(7):
            s = i * 7 + j
            flat_scr[:, s * C2:(s + 1) * C2] = p2r[i, j, :, :]
    h = jnp.dot(flat_scr[...], f1w_ref[...],
                preferred_element_type=jnp.float32)
    h = jnp.maximum(h + f1b_ref[...], 0.0)
    out_ref[...] = (jnp.dot(h, f2w_ref[...],
                            preferred_element_type=jnp.float32) + f2b_ref[...])


def kernel(x, conv1_w, conv1_b, conv2_w, conv2_b, fc1_w, fc1_b, fc2_w, fc2_b):
    B = x.shape[0]
    TB = 64
    while B % TB:
        TB //= 2

    xq = _quad_input(x)
    w1e = _conv1_effective_weight(conv1_w)
    b1c = jnp.tile(conv1_b.reshape(1, C1), (1, 16))
    w2g = _conv2_grouped_weight(conv2_w)
    b2c = jnp.tile(conv2_b.reshape(1, C2), (1, 4))
    f1w = _fc1_weight_nhwc(fc1_w)

    return pl.pallas_call(
        _fused_body,
        out_shape=jax.ShapeDtypeStruct((B, 10), jnp.float32),
        grid=(B // TB,),
        in_specs=[
            pl.BlockSpec((8, 8, TB, 16), lambda b: (0, 0, b, 0)),
            pl.BlockSpec((64, 16 * C1), lambda b: (0, 0)),
            pl.BlockSpec((1, 16 * C1), lambda b: (0, 0)),
            pl.BlockSpec((16 * C1, 4 * C2), lambda b: (0, 0)),
            pl.BlockSpec((1, 4 * C2), lambda b: (0, 0)),
            pl.BlockSpec((49 * C2, 32), lambda b: (0, 0)),
            pl.BlockSpec((1, 32), lambda b: (0, 0)),
            pl.BlockSpec((32, 10), lambda b: (0, 0)),
            pl.BlockSpec((1, 10), lambda b: (0, 0)),
        ],
        out_specs=pl.BlockSpec((TB, 10), lambda b: (b, 0)),
        scratch_shapes=[pltpu.VMEM((4, 8, 8, TB, C1), jnp.float32),
                        pltpu.VMEM((TB, 49 * C2), jnp.float32)],
        compiler_params=pltpu.CompilerParams(
            dimension_semantics=("parallel",),
            vmem_limit_bytes=64 * 1024 * 1024,
        ),
    )(xq, w1e, b1c, w2g, b2c,
      f1w, fc1_b.reshape(1, 32), fc2_w, fc2_b.reshape(1, 10))


# speedup vs baseline: 2.4389x; 1.1224x over previous
import numpy as np
import jax
import jax.numpy as jnp
from jax.experimental import pallas as pl
from jax.experimental.pallas import tpu as pltpu

C1 = 32
C2 = 16


def _quad_input(x):
    B = x.shape[0]
    xp = jnp.pad(x[:, 0], ((0, 0), (1, 3), (1, 3)))
    xq = xp.reshape(B, 8, 4, 8, 4).transpose(1, 3, 0, 2, 4)
    return xq.reshape(8, 8, B, 16)


def _conv1_effective_weight(conv1_w):
    sel = np.zeros((36, 16, 9), dtype=np.float32)
    for rp in (0, 1):
        for cp in (0, 1):
            for dy in (0, 1):
                for dx in (0, 1):
                    g = (dy * 2 + dx) * 4 + (rp * 2 + cp)
                    for ky in range(3):
                        for kx in range(3):
                            A = 2 * rp + dy + ky
                            e = 2 * cp + dx + kx
                            sel[A * 6 + e, g, ky * 3 + kx] = 1.0
    w1k = conv1_w.reshape(C1, 9).T
    w = jnp.einsum("tgk,kc->tgc", jnp.asarray(sel), w1k,
                   precision=jax.lax.Precision.HIGHEST)
    w = w.reshape(36, 16 * C1)
    qsel = np.zeros((4, 16, 36), dtype=np.float32)
    for dA in (0, 1):
        for dE in (0, 1):
            for a in range(4):
                for em in range(4):
                    A, e = 4 * dA + a, 4 * dE + em
                    if A < 6 and e < 6:
                        qsel[dA * 2 + dE, a * 4 + em, A * 6 + e] = 1.0
    wq = jnp.einsum("qst,tm->qsm", jnp.asarray(qsel), w,
                    precision=jax.lax.Precision.HIGHEST)
    return wq.reshape(64, 16 * C1)


def _conv2_grouped_weight(conv2_w):
    sel = np.zeros((16, 4, 9), dtype=np.float32)
    for u in range(4):
        for v in range(4):
            for dy in (0, 1):
                for dx in (0, 1):
                    ky, kx = u - dy, v - dx
                    if 0 <= ky < 3 and 0 <= kx < 3:
                        sel[u * 4 + v, dy * 2 + dx, ky * 3 + kx] = 1.0
    w2k = jnp.transpose(conv2_w, (2, 3, 1, 0)).reshape(9, C1, C2)
    w = jnp.einsum("sgk,kcd->scgd", jnp.asarray(sel), w2k,
                   precision=jax.lax.Precision.HIGHEST)
    return w.reshape(16 * C1, 4 * C2)


def _fc1_weight_nhwc(fc1_w):
    ii, jj, cc = np.meshgrid(np.arange(7), np.arange(7), np.arange(C2),
                             indexing="ij")
    perm = (cc * 49 + ii * 7 + jj).reshape(-1)
    return fc1_w[jnp.asarray(perm), :]


def _fused_body(xq_ref, w1_ref, b1_ref, w2_ref, b2_ref,
                f1w_ref, f1b_ref, f2w_ref, f2b_ref, out_ref, p1_scr,
                flat_scr):
    TB = xq_ref.shape[2]
    M = 49 * TB

    @pl.when(pl.program_id(0) == 0)
    def _init():
        p1_scr[...] = jnp.zeros_like(p1_scr)

    xq = xq_ref[...]
    t = jnp.concatenate(
        [xq[dA:dA + 7, dE:dE + 7, :, :]
         for dA in (0, 1) for dE in (0, 1)], axis=-1)
    z1 = jnp.dot(t.reshape(M, 64), w1_ref[...],
                 preferred_element_type=jnp.float32)
    z1 = jnp.maximum(z1 + b1_ref[...], 0.0)
    best4 = jnp.maximum(jnp.maximum(z1[:, 0:128], z1[:, 128:256]),
                        jnp.maximum(z1[:, 256:384], z1[:, 384:512]))
    for rp in (0, 1):
        for cp in (0, 1):
            best = best4[:, (rp * 2 + cp) * 32:(rp * 2 + cp + 1) * 32]
            p1_scr[(1 - rp) * 2 + (1 - cp), rp:rp + 7, cp:cp + 7, :, :] = (
                best.reshape(7, 7, TB, C1))

    q2 = jnp.concatenate(
        [p1_scr[(u % 2) * 2 + (v % 2),
                u // 2:u // 2 + 7, v // 2:v // 2 + 7, :, :]
         for u in range(4) for v in range(4)], axis=-1)
    z2 = jnp.dot(q2.reshape(M, 16 * C1), w2_ref[...],
                 preferred_element_type=jnp.float32)
    z2 = jnp.maximum(z2 + b2_ref[...], 0.0)
    p2 = jnp.maximum(jnp.maximum(z2[:, 0 * C2:1 * C2], z2[:, 1 * C2:2 * C2]),
                     jnp.maximum(z2[:, 2 * C2:3 * C2], z2[:, 3 * C2:4 * C2]))

    p2r = p2.reshape(7, 7, TB, C2)
    for i in range(7):
        for j in range(7):
            s = i * 7 + j
            flat_scr[:, s * C2:(s + 1) * C2] = p2r[i, j, :, :]
    h = jnp.dot(flat_scr[...], f1w_ref[...],
                preferred_element_type=jnp.float32)
    h = jnp.maximum(h + f1b_ref[...], 0.0)
    out_ref[...] = (jnp.dot(h, f2w_ref[...],
                            preferred_element_type=jnp.float32) + f2b_ref[...])


def kernel(x, conv1_w, conv1_b, conv2_w, conv2_b, fc1_w, fc1_b, fc2_w, fc2_b):
    B = x.shape[0]
    TB = 128
    while B % TB:
        TB //= 2

    xq = _quad_input(x)
    w1e = _conv1_effective_weight(conv1_w)
    b1c = jnp.tile(conv1_b.reshape(1, C1), (1, 16))
    w2g = _conv2_grouped_weight(conv2_w)
    b2c = jnp.tile(conv2_b.reshape(1, C2), (1, 4))
    f1w = _fc1_weight_nhwc(fc1_w)

    return pl.pallas_call(
        _fused_body,
        out_shape=jax.ShapeDtypeStruct((B, 10), jnp.float32),
        grid=(B // TB,),
        in_specs=[
            pl.BlockSpec((8, 8, TB, 16), lambda b: (0, 0, b, 0)),
            pl.BlockSpec((64, 16 * C1), lambda b: (0, 0)),
            pl.BlockSpec((1, 16 * C1), lambda b: (0, 0)),
            pl.BlockSpec((16 * C1, 4 * C2), lambda b: (0, 0)),
            pl.BlockSpec((1, 4 * C2), lambda b: (0, 0)),
            pl.BlockSpec((49 * C2, 32), lambda b: (0, 0)),
            pl.BlockSpec((1, 32), lambda b: (0, 0)),
            pl.BlockSpec((32, 10), lambda b: (0, 0)),
            pl.BlockSpec((1, 10), lambda b: (0, 0)),
        ],
        out_specs=pl.BlockSpec((TB, 10), lambda b: (b, 0)),
        scratch_shapes=[pltpu.VMEM((4, 8, 8, TB, C1), jnp.float32),
                        pltpu.VMEM((TB, 49 * C2), jnp.float32)],
        compiler_params=pltpu.CompilerParams(
            dimension_semantics=("parallel",),
            vmem_limit_bytes=64 * 1024 * 1024,
        ),
    )(xq, w1e, b1c, w2g, b2c,
      f1w, fc1_b.reshape(1, 32), fc2_w, fc2_b.reshape(1, 10))
